# Initial kernel scaffold; baseline (speedup 1.0000x reference)
import jax
import jax.numpy as jnp
from jax.experimental import pallas as pl


def kernel(x1, p1, x2, p2, W_up, gamma1, beta1, W_lat, gamma2, beta2):
    raise NotImplementedError("write your pallas kernel here")


# speedup vs baseline: 29.2352x; 29.2352x over previous
import functools

import jax
import jax.numpy as jnp
from jax import lax
from jax.experimental import pallas as pl
from jax.experimental.pallas import tpu as pltpu

B, N, M, CIN, COUT = 4, 1024, 4096, 256, 128
MB = 1024
EPS_BN = 1e-5
EPS_W = 1e-8


def _prep_kernel(x1_ref, x2_ref, wup_ref, g1_ref, b1_ref, wlat_ref, g2_ref,
                 b2_ref, h_ref, l_ref):
    wup = wup_ref[...]
    wlat = wlat_ref[...]

    s = jnp.zeros((COUT, 1), jnp.float32)
    ss = jnp.zeros((COUT, 1), jnp.float32)
    for b in range(B):
        hb = jnp.dot(wup, x1_ref[b], preferred_element_type=jnp.float32)
        h_ref[b] = hb
        s = s + jnp.sum(hb, axis=1, keepdims=True)
        ss = ss + jnp.sum(hb * hb, axis=1, keepdims=True)
    cnt = float(B * N)
    mean = s / cnt
    var = ss / cnt - mean * mean
    scale = g1_ref[...].reshape(COUT, 1) * lax.rsqrt(var + EPS_BN)
    shift = b1_ref[...].reshape(COUT, 1) - mean * scale
    for b in range(B):
        h_ref[b] = jnp.maximum(h_ref[b] * scale + shift, 0.0)

    s = jnp.zeros((COUT, 1), jnp.float32)
    ss = jnp.zeros((COUT, 1), jnp.float32)
    for b in range(B):
        lb = jnp.dot(wlat, x2_ref[b], preferred_element_type=jnp.float32)
        l_ref[b] = lb
        s = s + jnp.sum(lb, axis=1, keepdims=True)
        ss = ss + jnp.sum(lb * lb, axis=1, keepdims=True)
    cnt = float(B * M)
    mean = s / cnt
    var = ss / cnt - mean * mean
    scale = g2_ref[...].reshape(COUT, 1) * lax.rsqrt(var + EPS_BN)
    shift = b2_ref[...].reshape(COUT, 1) - mean * scale
    for b in range(B):
        l_ref[b] = jnp.maximum(l_ref[b] * scale + shift, 0.0)


def _knn_interp_kernel(p2_ref, p1_ref, h_ref, l_ref, out_ref):
    p2b = p2_ref[0]
    p1b = p1_ref[0]

    sqd = (
        jnp.sum(p2b * p2b, axis=1, keepdims=True)
        + jnp.sum(p1b * p1b, axis=1, keepdims=True).reshape(1, N)
        - 2.0 * lax.dot_general(p2b, p1b, (((1,), (1,)), ((), ())),
                                preferred_element_type=jnp.float32)
    )

    lane_iota = lax.broadcasted_iota(jnp.int32, (MB, N), 1)
    dists = []
    idxs = []
    for _ in range(3):
        d = jnp.min(sqd, axis=1, keepdims=True)
        cand = jnp.where(sqd == d, lane_iota, N)
        i = jnp.min(cand, axis=1, keepdims=True)
        sqd = jnp.where(lane_iota == i, jnp.float32(3.4e38), sqd)
        dists.append(d)
        idxs.append(i)

    recips = [1.0 / (d + EPS_W) for d in dists]
    norm = recips[0] + recips[1] + recips[2]
    weights = [r / norm for r in recips]

    row_iota = lax.broadcasted_iota(jnp.int32, (N, MB), 0)
    acc = jnp.zeros((N, MB), jnp.float32)
    for k in range(3):
        ik = idxs[k].reshape(1, MB)
        wk = weights[k].reshape(1, MB)
        acc = acc + jnp.where(row_iota == ik, wk, 0.0)

    interp = jnp.dot(h_ref[0], acc, preferred_element_type=jnp.float32)
    out_ref[0] = interp + l_ref[0]


@jax.jit
def kernel(x1, p1, x2, p2, W_up, gamma1, beta1, W_lat, gamma2, beta2):
    h, l = pl.pallas_call(
        _prep_kernel,
        out_shape=(
            jax.ShapeDtypeStruct((B, COUT, N), jnp.float32),
            jax.ShapeDtypeStruct((B, COUT, M), jnp.float32),
        ),
    )(x1, x2, W_up, gamma1, beta1, W_lat, gamma2, beta2)

    out = pl.pallas_call(
        _knn_interp_kernel,
        grid=(B, M // MB),
        in_specs=[
            pl.BlockSpec((1, MB, 3), lambda b, m: (b, m, 0)),
            pl.BlockSpec((1, N, 3), lambda b, m: (b, 0, 0)),
            pl.BlockSpec((1, COUT, N), lambda b, m: (b, 0, 0)),
            pl.BlockSpec((1, COUT, MB), lambda b, m: (b, 0, m)),
        ],
        out_specs=pl.BlockSpec((1, COUT, MB), lambda b, m: (b, 0, m)),
        out_shape=jax.ShapeDtypeStruct((B, COUT, M), jnp.float32),
    )(p2, p1, h, l)

    return (out, p2)
